# baseline (speedup 1.0000x reference)
import functools

import jax
import jax.numpy as jnp
from jax import lax
from jax.experimental import pallas as pl
from jax.experimental.pallas import tpu as pltpu
from jax.experimental.pallas import tpu_sc as plsc

_NC = 2
_NS = 16
_NW = _NC * _NS

_G = 128
_CHUNK = 128
_GPC = _CHUNK // _G
_NROW = 4
_NIDX = 4
_UNROLL = 4


@functools.cache
def _build_gather(B: int, V: int, D: int):
    assert B % (_NW * _CHUNK * _UNROLL) == 0
    b_per_w = B // _NW
    steps = b_per_w // _CHUNK
    rows_per_w = b_per_w // _G

    mesh = plsc.VectorSubcoreMesh(core_axis_name="c", subcore_axis_name="s")

    @functools.partial(
        pl.kernel,
        mesh=mesh,
        out_type=jax.ShapeDtypeStruct((B, D), jnp.float32),
        scratch_types=[
            pltpu.VMEM_SHARED((V, D), jnp.float32),
            pltpu.VMEM((_NIDX, _GPC, _G), jnp.int32),
            pltpu.VMEM((_NROW, _CHUNK, D), jnp.float32),
        ] + [pltpu.SemaphoreType.DMA] * (1 + _NROW + _NIDX),
    )
    def k(table_hbm, idx_hbm, out_hbm, table_sh, idx_v, rows_v, gsem, *sems):
        sid = lax.axis_index("s")
        wid = sid * _NC + lax.axis_index("c")
        idx_row0 = wid * rows_per_w
        out0 = wid * b_per_w
        wsems = sems[:_NROW]
        isems = sems[_NROW:]

        for s8 in range(8):
            off = s8 * 128
            size = min(128, V - off)

            @pl.when(sid == s8)
            def _(off=off, size=size):
                pltpu.sync_copy(
                    table_hbm.at[pl.ds(off, size)],
                    table_sh.at[pl.ds(off, size)],
                )

        plsc.subcore_barrier()

        def idx_fetch(g, ib):
            pltpu.async_copy(
                idx_hbm.at[pl.ds(idx_row0 + g * _GPC, _GPC)],
                idx_v.at[ib],
                isems[ib],
            )

        def step(g, rb, ib, wait_write, prefetch):
            buf_rows = rows_v.at[rb]
            if wait_write:
                pltpu.make_async_copy(
                    buf_rows, out_hbm.at[pl.ds(out0, _CHUNK)], wsems[rb]
                ).wait()
            pltpu.make_async_copy(
                idx_hbm.at[pl.ds(idx_row0, _GPC)], idx_v.at[ib], isems[ib]
            ).wait()
            copies = []
            for j in range(_GPC):
                copies.append(
                    pltpu.async_copy(
                        table_sh.at[idx_v.at[ib, j]],
                        buf_rows.at[pl.ds(j * _G, _G)],
                        gsem,
                    )
                )
            for c in copies:
                c.wait()
            if prefetch:
                idx_fetch(g + _NIDX, ib)
            pltpu.async_copy(
                buf_rows, out_hbm.at[pl.ds(out0 + g * _CHUNK, _CHUNK)],
                wsems[rb],
            )

        for p in range(_NIDX):
            idx_fetch(p, p)
        for p in range(_UNROLL):
            step(p, p % _NROW, p % _NIDX, p >= _NROW, True)

        def body(go, carry):
            g0 = go * _UNROLL
            for p in range(_UNROLL):
                step(g0 + p, p % _NROW, p % _NIDX, True, True)
            return carry

        lax.fori_loop(1, steps // _UNROLL - 1, body, 0)

        for p in range(_UNROLL):
            step(steps - _UNROLL + p, p % _NROW, p % _NIDX, True, False)

        for rb in range(_NROW):
            pltpu.make_async_copy(
                rows_v.at[rb], out_hbm.at[pl.ds(out0, _CHUNK)], wsems[rb]
            ).wait()

    return k


def kernel(time_gaps, pe):
    Rr, Cc = time_gaps.shape
    V, D = pe.shape[1], pe.shape[2]
    B = Rr * Cc
    idx = time_gaps.reshape(B // _G, _G).astype(jnp.int32)
    table = pe.reshape(V, D)
    out = _build_gather(B, V, D)(table, idx)
    return out.reshape(Rr, Cc, D)


# speedup vs baseline: 1.0746x; 1.0746x over previous
import functools

import jax
import jax.numpy as jnp
from jax import lax
from jax.experimental import pallas as pl
from jax.experimental.pallas import tpu as pltpu
from jax.experimental.pallas import tpu_sc as plsc

_NC = 2
_NS = 16
_NW = _NC * _NS

_G = 128
_CHUNK = 256
_GPC = _CHUNK // _G
_NROW = 2
_NIDX = 4
_UNROLL = 4


@functools.cache
def _build_gather(B: int, V: int, D: int):
    assert B % (_NW * _CHUNK * _UNROLL) == 0
    b_per_w = B // _NW
    steps = b_per_w // _CHUNK
    rows_per_w = b_per_w // _G
    _EPI = (steps - 1) % _UNROLL

    mesh = plsc.VectorSubcoreMesh(core_axis_name="c", subcore_axis_name="s")

    @functools.partial(
        pl.kernel,
        mesh=mesh,
        out_type=jax.ShapeDtypeStruct((B, D), jnp.float32),
        scratch_types=[
            pltpu.VMEM_SHARED((V, D), jnp.float32),
            pltpu.VMEM((_NIDX, _GPC, _G), jnp.int32),
            pltpu.VMEM((_NROW, _CHUNK, D), jnp.float32),
        ] + [pltpu.SemaphoreType.DMA] * (_NROW + _NROW + _NIDX),
    )
    def k(table_hbm, idx_hbm, out_hbm, table_sh, idx_v, rows_v, *sems):
        sid = lax.axis_index("s")
        wid = sid * _NC + lax.axis_index("c")
        idx_row0 = wid * rows_per_w
        out0 = wid * b_per_w
        gsems = sems[:_NROW]
        wsems = sems[_NROW:2 * _NROW]
        isems = sems[2 * _NROW:]

        for s8 in range(8):
            off = s8 * 128
            size = min(128, V - off)

            @pl.when(sid == s8)
            def _(off=off, size=size):
                pltpu.sync_copy(
                    table_hbm.at[pl.ds(off, size)],
                    table_sh.at[pl.ds(off, size)],
                )

        plsc.subcore_barrier()

        def idx_fetch(g, ib):
            pltpu.async_copy(
                idx_hbm.at[pl.ds(idx_row0 + g * _GPC, _GPC)],
                idx_v.at[ib],
                isems[ib],
            )

        def wait_idx(ib):
            pltpu.make_async_copy(
                idx_hbm.at[pl.ds(idx_row0, _GPC)], idx_v.at[ib], isems[ib]
            ).wait()

        def wait_write(rb):
            pltpu.make_async_copy(
                rows_v.at[rb], out_hbm.at[pl.ds(out0, _CHUNK)], wsems[rb]
            ).wait()

        def issue_gathers(g, rb, ib):
            for j in range(_GPC):
                pltpu.async_copy(
                    table_sh.at[idx_v.at[ib, j]],
                    rows_v.at[rb].at[pl.ds(j * _G, _G)],
                    gsems[rb],
                )

        def drain_gathers(rb):
            for j in range(_GPC):
                pltpu.make_async_copy(
                    table_hbm.at[pl.ds(0, _G)],
                    rows_v.at[rb].at[pl.ds(j * _G, _G)],
                    gsems[rb],
                ).wait()

        def issue_write(g, rb):
            pltpu.async_copy(
                rows_v.at[rb],
                out_hbm.at[pl.ds(out0 + g * _CHUNK, _CHUNK)],
                wsems[rb],
            )

        def slot(g, p, first=False, prefetch=True, wait_w=True):
            rb, ib = p % _NROW, p % _NIDX
            prb, pib = (p - 1) % _NROW, (p - 1) % _NIDX
            if not first:
                if wait_w:
                    wait_write(rb)
                wait_idx(ib)
                issue_gathers(g, rb, ib)
                drain_gathers(prb)
                if prefetch:
                    idx_fetch(g + _NIDX - 1, pib)
                issue_write(g - 1, prb)
            else:
                wait_idx(ib)
                issue_gathers(g, rb, ib)

        for p in range(_NIDX):
            idx_fetch(p, p)
        slot(0, 0, first=True)
        slot(1, 1, wait_w=False)

        def body(go, carry):
            g0 = go * _UNROLL + 2
            for i in range(_UNROLL):
                slot(g0 + i, (2 + i) % _UNROLL)
            return carry

        n_body = (steps - 2 - (_EPI + _UNROLL - 1)) // _UNROLL
        lax.fori_loop(0, n_body, body, 0)

        for g in range(2 + n_body * _UNROLL, steps):
            slot(g, g % _UNROLL, prefetch=(g + _NIDX - 1 < steps))

        last = steps - 1
        drain_gathers(last % _NROW)
        issue_write(last, last % _NROW)
        for rb in range(_NROW):
            wait_write(rb)

    return k


def kernel(time_gaps, pe):
    Rr, Cc = time_gaps.shape
    V, D = pe.shape[1], pe.shape[2]
    B = Rr * Cc
    idx = time_gaps.reshape(B // _G, _G).astype(jnp.int32)
    table = pe.reshape(V, D)
    out = _build_gather(B, V, D)(table, idx)
    return out.reshape(Rr, Cc, D)
